# baseline (speedup 1.0000x reference)
import functools

import jax
import jax.numpy as jnp
from jax import lax
from jax.experimental import pallas as pl
from jax.experimental.pallas import tpu as pltpu
from jax.experimental.pallas import tpu_sc as plsc

N = 10000
E = 320000
DH = 128
G = 64
C = 19

NC = 2
NS = 16
B = 128
NB = 160
NBUF = 8
KLA = NBUF - 2
ZB = 64
EP = NS * B * NB
HD = DH // NC
QW = 32
NQ = 4
NPAD = 10240
ROWS_PER_TILE = NPAD // NS
RCH = ROWS_PER_TILE // B

_sc_mesh = plsc.VectorSubcoreMesh(
    core_axis_name="c", subcore_axis_name="s", num_cores=NC, num_subcores=NS)


def _fill_rows(ref, nrows, width, value):
  vec = jnp.full((16,), value, jnp.float32)

  def row(i, _):
    for k in range(width // 16):
      ref[i, pl.ds(k * 16, 16)] = vec
    return 0

  lax.fori_loop(0, nrows, row, 0, unroll=False)


def _make_sc_agg():
  out_type = [jax.ShapeDtypeStruct((NQ, NPAD, QW), jnp.float32)]

  scratch = [
      pltpu.VMEM((NB + NBUF, B), jnp.int32),
      pltpu.VMEM((NB, B), jnp.int32),
      [pltpu.VMEM((B, QW), jnp.float32) for _ in range(NBUF)],
      pltpu.VMEM((ZB, QW), jnp.float32),
      pltpu.VMEM_SHARED((NPAD, QW), jnp.float32),
      pltpu.VMEM_SHARED((NPAD, QW), jnp.float32),
      [pltpu.SemaphoreType.DMA for _ in range(NBUF)],
      [pltpu.SemaphoreType.DMA for _ in range(NBUF)],
  ]

  def body(m4_hbm, src_hbm, dst_hbm, out_hbm,
           srcv, dstv, bufs, zbuf, mS, aggS, gsems, ssems):
    c = lax.axis_index("c")
    s = lax.axis_index("s")

    pltpu.sync_copy(src_hbm.at[s], srcv)
    pltpu.sync_copy(dst_hbm.at[s], dstv)
    _fill_rows(zbuf, ZB, QW, 0.0)

    def gather(j, bi):
      pltpu.make_async_copy(mS.at[srcv.at[j]], bufs[bi], gsems[bi]).start()

    def gwait(j, bi):
      pltpu.make_async_copy(mS.at[srcv.at[j]], bufs[bi], gsems[bi]).wait()

    def sstart(j, bi):
      pltpu.async_copy(bufs[bi], aggS.at[dstv.at[j]], ssems[bi], add=True)

    def swait(bi):
      pltpu.make_async_copy(bufs[bi], aggS.at[dstv.at[0]], ssems[bi]).wait()

    def stepj(j, bi, first):
      gwait(j, bi)
      sstart(j, bi)
      sw = (bi + KLA) % NBUF
      if not first:
        swait(sw)
      gather(j + KLA, sw)

    for p in range(2):
      qc = 2 * p + c
      for t in range(ROWS_PER_TILE // ZB):
        pltpu.sync_copy(zbuf, aggS.at[pl.ds(s * ROWS_PER_TILE + t * ZB, ZB)])
      pltpu.sync_copy(m4_hbm.at[qc, pl.ds(s * ROWS_PER_TILE, ROWS_PER_TILE)],
                      mS.at[pl.ds(s * ROWS_PER_TILE, ROWS_PER_TILE)])
      plsc.subcore_barrier()

      for j in range(KLA):
        gather(j, j)
      for j in range(NBUF):
        stepj(j, j, j < NBUF - KLA)

      def step(g, _):
        for bi in range(NBUF):
          stepj(g * NBUF + bi, bi, False)
        return 0

      lax.fori_loop(1, NB // NBUF, step, 0, unroll=False)
      for i in range(KLA):
        gwait(NB + i, i)
      swait(NBUF - 2)
      swait(NBUF - 1)
      plsc.subcore_barrier()

      for t in range(RCH):
        r = (s * RCH + t) * B
        pltpu.sync_copy(aggS.at[pl.ds(r, B)], out_hbm.at[qc, pl.ds(r, B)])

  return pl.kernel(
      body, out_type=out_type, mesh=_sc_mesh, scratch_types=scratch,
      compiler_params=pltpu.CompilerParams(use_tc_tiling_on_sc=False))


NBD = EP // (NC * NS * B)


def _make_sc_deg():
  out_type = [jax.ShapeDtypeStruct((NC, NPAD, 16), jnp.float32)]
  scratch = [
      pltpu.VMEM((NBD, B), jnp.int32),
      pltpu.VMEM((B, 16), jnp.float32),
      pltpu.VMEM((B, 16), jnp.float32),
      pltpu.VMEM_SHARED((NPAD, 16), jnp.float32),
  ]

  def body(dst_hbm, out_hbm, dstv, onesv, z16, degS):
    c = lax.axis_index("c")
    s = lax.axis_index("s")
    w = c * NS + s

    pltpu.sync_copy(dst_hbm.at[w], dstv)
    _fill_rows(onesv, B, 16, 1.0)
    _fill_rows(z16, B, 16, 0.0)
    for t in range(RCH):
      pltpu.sync_copy(z16, degS.at[pl.ds((s * RCH + t) * B, B)])
    plsc.subcore_barrier()

    def step(j, _):
      pltpu.sync_copy(onesv, degS.at[dstv.at[j]], add=True)
      return 0

    lax.fori_loop(0, NBD, step, 0, unroll=False)
    plsc.subcore_barrier()

    for t in range(RCH):
      r = (s * RCH + t) * B
      pltpu.sync_copy(degS.at[pl.ds(r, B)], out_hbm.at[c, pl.ds(r, B)])

  return pl.kernel(
      body, out_type=out_type, mesh=_sc_mesh, scratch_types=scratch,
      compiler_params=pltpu.CompilerParams(use_tc_tiling_on_sc=False))


_sc_agg = _make_sc_agg()
_sc_deg = _make_sc_deg()

BLK = 1000
NBLK = N // BLK


def _tc1_body(x_ref, wn_ref, ws_ref, m_ref, s_ref):
  xb = x_ref[...]
  m = jnp.dot(xb, wn_ref[...], preferred_element_type=jnp.float32)
  for q in range(NQ):
    m_ref[q] = m[:, q * QW:(q + 1) * QW]
  s_ref[...] = jnp.dot(xb, ws_ref[...], preferred_element_type=jnp.float32)


def _tc1(x, wn, ws):
  return pl.pallas_call(
      _tc1_body,
      grid=(NBLK,),
      in_specs=[
          pl.BlockSpec((BLK, DH), lambda i: (i, 0)),
          pl.BlockSpec((DH, DH), lambda i: (0, 0)),
          pl.BlockSpec((DH, DH), lambda i: (0, 0)),
      ],
      out_specs=[
          pl.BlockSpec((NQ, BLK, QW), lambda i: (0, i, 0)),
          pl.BlockSpec((BLK, DH), lambda i: (i, 0)),
      ],
      out_shape=[
          jax.ShapeDtypeStruct((NQ, NPAD, QW), jnp.float32),
          jax.ShapeDtypeStruct((N, DH), jnp.float32),
      ],
  )(x, wn, ws)


def _combine(s_ref, a_ref, d_ref):
  agg = jnp.concatenate([a_ref[0], a_ref[1], a_ref[2], a_ref[3]], axis=1)
  deg = d_ref[0, :, 0:1] + d_ref[1, :, 0:1]
  h = s_ref[...] + agg / jnp.maximum(deg, 1.0)
  return jnp.where(h > 0, h, jnp.exp(h) - 1.0)


def _tc2_body(s_ref, a_ref, d_ref, wn_ref, ws_ref, m_ref, o_ref):
  h = _combine(s_ref, a_ref, d_ref)
  m = jnp.dot(h, wn_ref[...], preferred_element_type=jnp.float32)
  for q in range(NQ):
    m_ref[q] = m[:, q * QW:(q + 1) * QW]
  o_ref[...] = jnp.dot(h, ws_ref[...], preferred_element_type=jnp.float32)


def _tc2(s1, agg1, degw, wn, ws):
  return pl.pallas_call(
      _tc2_body,
      grid=(NBLK,),
      in_specs=[
          pl.BlockSpec((BLK, DH), lambda i: (i, 0)),
          pl.BlockSpec((NQ, BLK, QW), lambda i: (0, i, 0)),
          pl.BlockSpec((NC, BLK, 16), lambda i: (0, i, 0)),
          pl.BlockSpec((DH, DH), lambda i: (0, 0)),
          pl.BlockSpec((DH, DH), lambda i: (0, 0)),
      ],
      out_specs=[
          pl.BlockSpec((NQ, BLK, QW), lambda i: (0, i, 0)),
          pl.BlockSpec((BLK, DH), lambda i: (i, 0)),
      ],
      out_shape=[
          jax.ShapeDtypeStruct((NQ, NPAD, QW), jnp.float32),
          jax.ShapeDtypeStruct((N, DH), jnp.float32),
      ],
  )(s1, agg1, degw, wn, ws)


def _tc3_body(gid_ref, s_ref, a_ref, d_ref, emb_ref, sent_ref):
  i = pl.program_id(0)
  emb = _combine(s_ref, a_ref, d_ref)
  emb_ref[...] = emb

  @pl.when(i == 0)
  def _():
    sent_ref[...] = jnp.full((G, DH), -jnp.inf, jnp.float32)

  gid = gid_ref[...]
  gmin = jnp.min(gid)
  gmax = jnp.max(gid)

  def gbody(g, _):
    v = jnp.max(jnp.where(gid == g, emb, -jnp.inf), axis=0, keepdims=True)
    cur = sent_ref[pl.ds(g, 1), :]
    sent_ref[pl.ds(g, 1), :] = jnp.maximum(cur, v)
    return 0

  lax.fori_loop(gmin, gmax + 1, gbody, 0, unroll=False)


def _tc3(gid2d, s2, agg2, degw):
  return pl.pallas_call(
      _tc3_body,
      grid=(NBLK,),
      in_specs=[
          pl.BlockSpec((BLK, 1), lambda i: (i, 0)),
          pl.BlockSpec((BLK, DH), lambda i: (i, 0)),
          pl.BlockSpec((NQ, BLK, QW), lambda i: (0, i, 0)),
          pl.BlockSpec((NC, BLK, 16), lambda i: (0, i, 0)),
      ],
      out_specs=[
          pl.BlockSpec((BLK, DH), lambda i: (i, 0)),
          pl.BlockSpec((G, DH), lambda i: (0, 0)),
      ],
      out_shape=[
          jax.ShapeDtypeStruct((N, DH), jnp.float32),
          jax.ShapeDtypeStruct((G, DH), jnp.float32),
      ],
  )(gid2d, s2, agg2, degw)


def _tc4_body(e1_ref, e2_ref, sent_ref, emb_ref, tgt_ref, wl_ref, bl_ref,
              pred_ref, loss_ref):
  emb = emb_ref[...]
  nodes = lax.broadcasted_iota(jnp.int32, (G, N), 1)
  oh1 = (nodes == e1_ref[...]).astype(jnp.float32)
  oh2 = (nodes == e2_ref[...]).astype(jnp.float32)
  e1rows = jnp.dot(oh1, emb, preferred_element_type=jnp.float32)
  e2rows = jnp.dot(oh2, emb, preferred_element_type=jnp.float32)
  h = jnp.concatenate([sent_ref[...], e1rows, e2rows], axis=1)
  logits = jnp.dot(h, wl_ref[...], preferred_element_type=jnp.float32)
  logits = logits + bl_ref[...]
  mx = jnp.max(logits, axis=1, keepdims=True)
  lse = mx + jnp.log(jnp.sum(jnp.exp(logits - mx), axis=1, keepdims=True))
  logp = logits - lse
  cols = lax.broadcasted_iota(jnp.int32, (G, C), 1)
  onehot = cols == tgt_ref[...]
  picked = jnp.sum(jnp.where(onehot, logp, 0.0), axis=1)
  ce = -jnp.mean(picked)
  loss = ce + 0.003 * jnp.mean(jnp.sum(h * h, axis=1))
  loss_ref[...] = jnp.reshape(loss, (1, 1))
  pred = jnp.min(jnp.where(logits == mx, cols, C), axis=1)
  pred_ref[...] = pred.reshape(G, 1)


def _tc4(e1_idx, e2_idx, sent, emb, tgt2d, wl, bl2d):
  return pl.pallas_call(
      _tc4_body,
      in_specs=[
          pl.BlockSpec((G, 1), lambda: (0, 0)),
          pl.BlockSpec((G, 1), lambda: (0, 0)),
          pl.BlockSpec((G, DH), lambda: (0, 0)),
          pl.BlockSpec((N, DH), lambda: (0, 0)),
          pl.BlockSpec((G, 1), lambda: (0, 0)),
          pl.BlockSpec((3 * DH, C), lambda: (0, 0)),
          pl.BlockSpec((1, C), lambda: (0, 0)),
      ],
      out_specs=[
          pl.BlockSpec((G, 1), lambda: (0, 0)),
          pl.BlockSpec((1, 1), lambda: (0, 0)),
      ],
      out_shape=[
          jax.ShapeDtypeStruct((G, 1), jnp.int32),
          jax.ShapeDtypeStruct((1, 1), jnp.float32),
      ],
  )(e1_idx, e2_idx, sent, emb, tgt2d, wl, bl2d)


def kernel(x, edge_index, graph_ids, e1_idx, e2_idx, tgt,
           W_self1, W_neigh1, W_self2, W_neigh2, W_lin, b_lin):
  src = edge_index[0]
  dst = edge_index[1]
  pad = EP - E
  srcp = jnp.concatenate([src, jnp.zeros((pad,), jnp.int32)])
  dstp = jnp.concatenate([dst, jnp.full((pad,), N, jnp.int32)])
  src3 = jnp.concatenate(
      [srcp.reshape(NS, NB, B),
       jnp.zeros((NS, NBUF, B), jnp.int32)], axis=1)
  dst3 = dstp.reshape(NS, NB, B)
  dst4 = dstp.reshape(NC * NS, NBD, B)
  gid2d = graph_ids.reshape(N, 1)
  tgt2d = tgt.reshape(G, 1).astype(jnp.int32)
  bl2d = b_lin.reshape(1, C)

  m1, s1 = _tc1(x, W_neigh1, W_self1)
  degw, = _sc_deg(dst4)
  agg1, = _sc_agg(m1, src3, dst3)
  m2, s2 = _tc2(s1, agg1, degw, W_neigh2, W_self2)
  agg2, = _sc_agg(m2, src3, dst3)
  emb, sent = _tc3(gid2d, s2, agg2, degw)
  pred2, loss2 = _tc4(e1_idx.reshape(G, 1), e2_idx.reshape(G, 1),
                      sent, emb, tgt2d, W_lin, bl2d)
  return pred2[:, 0], loss2[0, 0]


# speedup vs baseline: 1.1720x; 1.1720x over previous
import functools

import jax
import jax.numpy as jnp
from jax import lax
from jax.experimental import pallas as pl
from jax.experimental.pallas import tpu as pltpu
from jax.experimental.pallas import tpu_sc as plsc

N = 10000
E = 320000
DH = 128
G = 64
C = 19

NC = 2
NS = 16
B = 128
NB = 160
NBUF = 5
KLA = NBUF - 2
ZB = 64
EP = NS * B * NB
HD = DH // NC
QW = 32
NQ = 4
NPAD = 10240
ROWS_PER_TILE = NPAD // NS
RCH = ROWS_PER_TILE // B

_sc_mesh = plsc.VectorSubcoreMesh(
    core_axis_name="c", subcore_axis_name="s", num_cores=NC, num_subcores=NS)


def _fill_rows(ref, nrows, width, value):
  vec = jnp.full((16,), value, jnp.float32)

  def row(i, _):
    for k in range(width // 16):
      ref[i, pl.ds(k * 16, 16)] = vec
    return 0

  lax.fori_loop(0, nrows, row, 0, unroll=False)


def _make_sc_agg():
  out_type = [jax.ShapeDtypeStruct((NPAD, DH), jnp.float32)]

  scratch = [
      pltpu.VMEM((NB + NBUF, B), jnp.int32),
      pltpu.VMEM((NB, B), jnp.int32),
      [pltpu.VMEM((B, QW), jnp.float32) for _ in range(NBUF)],
      pltpu.VMEM((ZB, QW), jnp.float32),
      pltpu.VMEM_SHARED((NPAD, QW), jnp.float32),
      pltpu.VMEM_SHARED((NPAD, QW), jnp.float32),
      [pltpu.SemaphoreType.DMA for _ in range(NBUF)],
      [pltpu.SemaphoreType.DMA for _ in range(NBUF)],
  ]

  def body(m4_hbm, src_hbm, dst_hbm, out_hbm,
           srcv, dstv, bufs, zbuf, mS, aggS, gsems, ssems):
    c = lax.axis_index("c")
    s = lax.axis_index("s")

    pltpu.sync_copy(src_hbm.at[s], srcv)
    pltpu.sync_copy(dst_hbm.at[s], dstv)
    _fill_rows(zbuf, ZB, QW, 0.0)

    def gather(j, bi):
      pltpu.make_async_copy(mS.at[srcv.at[j]], bufs[bi], gsems[bi]).start()

    def gwait(j, bi):
      pltpu.make_async_copy(mS.at[srcv.at[j]], bufs[bi], gsems[bi]).wait()

    def sstart(j, bi):
      pltpu.async_copy(bufs[bi], aggS.at[dstv.at[j]], ssems[bi], add=True)

    def swait(bi):
      pltpu.make_async_copy(bufs[bi], aggS.at[dstv.at[0]], ssems[bi]).wait()

    def stepj(j, bi, first):
      gwait(j, bi)
      sstart(j, bi)
      sw = (bi + KLA) % NBUF
      if not first:
        swait(sw)
      gather(j + KLA, sw)

    for p in range(2):
      qc = 2 * p + c
      for t in range(ROWS_PER_TILE // ZB):
        pltpu.sync_copy(zbuf, aggS.at[pl.ds(s * ROWS_PER_TILE + t * ZB, ZB)])
      pltpu.sync_copy(m4_hbm.at[qc, pl.ds(s * ROWS_PER_TILE, ROWS_PER_TILE)],
                      mS.at[pl.ds(s * ROWS_PER_TILE, ROWS_PER_TILE)])
      plsc.subcore_barrier()

      for j in range(KLA):
        gather(j, j)
      for j in range(NBUF):
        stepj(j, j, j < NBUF - KLA)

      def step(g, _):
        for bi in range(NBUF):
          stepj(g * NBUF + bi, bi, False)
        return 0

      lax.fori_loop(1, NB // NBUF, step, 0, unroll=False)
      for i in range(KLA):
        gwait(NB + i, i)
      swait(NBUF - 2)
      swait(NBUF - 1)
      plsc.subcore_barrier()

      for t in range(RCH):
        r = (s * RCH + t) * B
        pltpu.sync_copy(aggS.at[pl.ds(r, B)],
                        out_hbm.at[pl.ds(r, B), pl.ds(qc * QW, QW)])

  return pl.kernel(
      body, out_type=out_type, mesh=_sc_mesh, scratch_types=scratch,
      compiler_params=pltpu.CompilerParams(use_tc_tiling_on_sc=False))


NBD = EP // (NC * NS * B)


def _make_sc_deg():
  out_type = [jax.ShapeDtypeStruct((NC, NPAD, 16), jnp.float32)]
  scratch = [
      pltpu.VMEM((NBD, B), jnp.int32),
      pltpu.VMEM((B, 16), jnp.float32),
      pltpu.VMEM((B, 16), jnp.float32),
      pltpu.VMEM_SHARED((NPAD, 16), jnp.float32),
  ]

  def body(dst_hbm, out_hbm, dstv, onesv, z16, degS):
    c = lax.axis_index("c")
    s = lax.axis_index("s")
    w = c * NS + s

    pltpu.sync_copy(dst_hbm.at[w], dstv)
    _fill_rows(onesv, B, 16, 1.0)
    _fill_rows(z16, B, 16, 0.0)
    for t in range(RCH):
      pltpu.sync_copy(z16, degS.at[pl.ds((s * RCH + t) * B, B)])
    plsc.subcore_barrier()

    def step(j, _):
      pltpu.sync_copy(onesv, degS.at[dstv.at[j]], add=True)
      return 0

    lax.fori_loop(0, NBD, step, 0, unroll=False)
    plsc.subcore_barrier()

    for t in range(RCH):
      r = (s * RCH + t) * B
      pltpu.sync_copy(degS.at[pl.ds(r, B)], out_hbm.at[c, pl.ds(r, B)])

  return pl.kernel(
      body, out_type=out_type, mesh=_sc_mesh, scratch_types=scratch,
      compiler_params=pltpu.CompilerParams(use_tc_tiling_on_sc=False))


_sc_agg = _make_sc_agg()
_sc_deg = _make_sc_deg()

BLK = 1000
NBLK = N // BLK


def _tc1_body(x_ref, wn_ref, ws_ref, m_ref, s_ref):
  xb = x_ref[...]
  m = jnp.dot(xb, wn_ref[...], preferred_element_type=jnp.float32)
  for q in range(NQ):
    m_ref[q] = m[:, q * QW:(q + 1) * QW]
  s_ref[...] = jnp.dot(xb, ws_ref[...], preferred_element_type=jnp.float32)


def _tc1(x, wn, ws):
  return pl.pallas_call(
      _tc1_body,
      grid=(NBLK,),
      in_specs=[
          pl.BlockSpec((BLK, DH), lambda i: (i, 0)),
          pl.BlockSpec((DH, DH), lambda i: (0, 0)),
          pl.BlockSpec((DH, DH), lambda i: (0, 0)),
      ],
      out_specs=[
          pl.BlockSpec((NQ, BLK, QW), lambda i: (0, i, 0)),
          pl.BlockSpec((BLK, DH), lambda i: (i, 0)),
      ],
      out_shape=[
          jax.ShapeDtypeStruct((NQ, NPAD, QW), jnp.float32),
          jax.ShapeDtypeStruct((N, DH), jnp.float32),
      ],
  )(x, wn, ws)


def _combine(s_ref, a_ref, d_ref):
  deg = d_ref[0, :, 0:1] + d_ref[1, :, 0:1]
  h = s_ref[...] + a_ref[...] / jnp.maximum(deg, 1.0)
  return jnp.where(h > 0, h, jnp.exp(h) - 1.0)


def _tc2_body(s_ref, a_ref, d_ref, wn_ref, ws_ref, m_ref, o_ref):
  h = _combine(s_ref, a_ref, d_ref)
  m = jnp.dot(h, wn_ref[...], preferred_element_type=jnp.float32)
  for q in range(NQ):
    m_ref[q] = m[:, q * QW:(q + 1) * QW]
  o_ref[...] = jnp.dot(h, ws_ref[...], preferred_element_type=jnp.float32)


def _tc2(s1, agg1, degw, wn, ws):
  return pl.pallas_call(
      _tc2_body,
      grid=(NBLK,),
      in_specs=[
          pl.BlockSpec((BLK, DH), lambda i: (i, 0)),
          pl.BlockSpec((BLK, DH), lambda i: (i, 0)),
          pl.BlockSpec((NC, BLK, 16), lambda i: (0, i, 0)),
          pl.BlockSpec((DH, DH), lambda i: (0, 0)),
          pl.BlockSpec((DH, DH), lambda i: (0, 0)),
      ],
      out_specs=[
          pl.BlockSpec((NQ, BLK, QW), lambda i: (0, i, 0)),
          pl.BlockSpec((BLK, DH), lambda i: (i, 0)),
      ],
      out_shape=[
          jax.ShapeDtypeStruct((NQ, NPAD, QW), jnp.float32),
          jax.ShapeDtypeStruct((N, DH), jnp.float32),
      ],
  )(s1, agg1, degw, wn, ws)


def _tc3_body(gid_ref, s_ref, a_ref, d_ref, emb_ref, sent_ref):
  i = pl.program_id(0)
  emb = _combine(s_ref, a_ref, d_ref)
  emb_ref[...] = emb

  @pl.when(i == 0)
  def _():
    sent_ref[...] = jnp.full((G, DH), -jnp.inf, jnp.float32)

  gid = gid_ref[...]
  gmin = jnp.min(gid)
  gmax = jnp.max(gid)

  def gbody(g, _):
    v = jnp.max(jnp.where(gid == g, emb, -jnp.inf), axis=0, keepdims=True)
    cur = sent_ref[pl.ds(g, 1), :]
    sent_ref[pl.ds(g, 1), :] = jnp.maximum(cur, v)
    return 0

  lax.fori_loop(gmin, gmax + 1, gbody, 0, unroll=False)


def _tc3(gid2d, s2, agg2, degw):
  return pl.pallas_call(
      _tc3_body,
      grid=(NBLK,),
      in_specs=[
          pl.BlockSpec((BLK, 1), lambda i: (i, 0)),
          pl.BlockSpec((BLK, DH), lambda i: (i, 0)),
          pl.BlockSpec((BLK, DH), lambda i: (i, 0)),
          pl.BlockSpec((NC, BLK, 16), lambda i: (0, i, 0)),
      ],
      out_specs=[
          pl.BlockSpec((BLK, DH), lambda i: (i, 0)),
          pl.BlockSpec((G, DH), lambda i: (0, 0)),
      ],
      out_shape=[
          jax.ShapeDtypeStruct((N, DH), jnp.float32),
          jax.ShapeDtypeStruct((G, DH), jnp.float32),
      ],
  )(gid2d, s2, agg2, degw)


def _tc4_body(e1_ref, e2_ref, sent_ref, emb_ref, tgt_ref, wl_ref, bl_ref,
              pred_ref, loss_ref):
  emb = emb_ref[...]
  nodes = lax.broadcasted_iota(jnp.int32, (G, N), 1)
  oh1 = (nodes == e1_ref[...]).astype(jnp.float32)
  oh2 = (nodes == e2_ref[...]).astype(jnp.float32)
  e1rows = jnp.dot(oh1, emb, preferred_element_type=jnp.float32)
  e2rows = jnp.dot(oh2, emb, preferred_element_type=jnp.float32)
  h = jnp.concatenate([sent_ref[...], e1rows, e2rows], axis=1)
  logits = jnp.dot(h, wl_ref[...], preferred_element_type=jnp.float32)
  logits = logits + bl_ref[...]
  mx = jnp.max(logits, axis=1, keepdims=True)
  lse = mx + jnp.log(jnp.sum(jnp.exp(logits - mx), axis=1, keepdims=True))
  logp = logits - lse
  cols = lax.broadcasted_iota(jnp.int32, (G, C), 1)
  onehot = cols == tgt_ref[...]
  picked = jnp.sum(jnp.where(onehot, logp, 0.0), axis=1)
  ce = -jnp.mean(picked)
  loss = ce + 0.003 * jnp.mean(jnp.sum(h * h, axis=1))
  loss_ref[...] = jnp.reshape(loss, (1, 1))
  pred = jnp.min(jnp.where(logits == mx, cols, C), axis=1)
  pred_ref[...] = pred.reshape(G, 1)


def _tc4(e1_idx, e2_idx, sent, emb, tgt2d, wl, bl2d):
  return pl.pallas_call(
      _tc4_body,
      in_specs=[
          pl.BlockSpec((G, 1), lambda: (0, 0)),
          pl.BlockSpec((G, 1), lambda: (0, 0)),
          pl.BlockSpec((G, DH), lambda: (0, 0)),
          pl.BlockSpec((N, DH), lambda: (0, 0)),
          pl.BlockSpec((G, 1), lambda: (0, 0)),
          pl.BlockSpec((3 * DH, C), lambda: (0, 0)),
          pl.BlockSpec((1, C), lambda: (0, 0)),
      ],
      out_specs=[
          pl.BlockSpec((G, 1), lambda: (0, 0)),
          pl.BlockSpec((1, 1), lambda: (0, 0)),
      ],
      out_shape=[
          jax.ShapeDtypeStruct((G, 1), jnp.int32),
          jax.ShapeDtypeStruct((1, 1), jnp.float32),
      ],
  )(e1_idx, e2_idx, sent, emb, tgt2d, wl, bl2d)


def kernel(x, edge_index, graph_ids, e1_idx, e2_idx, tgt,
           W_self1, W_neigh1, W_self2, W_neigh2, W_lin, b_lin):
  src = edge_index[0]
  dst = edge_index[1]
  pad = EP - E
  srcp = jnp.concatenate([src, jnp.zeros((pad,), jnp.int32)])
  dstp = jnp.concatenate([dst, jnp.full((pad,), N, jnp.int32)])
  src3 = jnp.concatenate(
      [srcp.reshape(NS, NB, B),
       jnp.zeros((NS, NBUF, B), jnp.int32)], axis=1)
  dst3 = dstp.reshape(NS, NB, B)
  dst4 = dstp.reshape(NC * NS, NBD, B)
  gid2d = graph_ids.reshape(N, 1)
  tgt2d = tgt.reshape(G, 1).astype(jnp.int32)
  bl2d = b_lin.reshape(1, C)

  m1, s1 = _tc1(x, W_neigh1, W_self1)
  degw, = _sc_deg(dst4)
  agg1, = _sc_agg(m1, src3, dst3)
  m2, s2 = _tc2(s1, agg1, degw, W_neigh2, W_self2)
  agg2, = _sc_agg(m2, src3, dst3)
  emb, sent = _tc3(gid2d, s2, agg2, degw)
  pred2, loss2 = _tc4(e1_idx.reshape(G, 1), e2_idx.reshape(G, 1),
                      sent, emb, tgt2d, W_lin, bl2d)
  return pred2[:, 0], loss2[0, 0]


# speedup vs baseline: 1.2371x; 1.0556x over previous
import functools

import jax
import jax.numpy as jnp
from jax import lax
from jax.experimental import pallas as pl
from jax.experimental.pallas import tpu as pltpu
from jax.experimental.pallas import tpu_sc as plsc

N = 10000
E = 320000
DH = 128
G = 64
C = 19

NC = 2
NS = 16
B = 128
NB = 160
NBUF = 5
KLA = NBUF - 2
ZB = 64
EP = NS * B * NB
HD = DH // NC
QW = 32
NQ = 4
NPAD = 10240
ROWS_PER_TILE = NPAD // NS
RCH = ROWS_PER_TILE // B

_sc_mesh = plsc.VectorSubcoreMesh(
    core_axis_name="c", subcore_axis_name="s", num_cores=NC, num_subcores=NS)


def _fill_rows(ref, nrows, width, value):
  vec = jnp.full((16,), value, jnp.float32)

  def row(i, _):
    for k in range(width // 16):
      ref[i, pl.ds(k * 16, 16)] = vec
    return 0

  lax.fori_loop(0, nrows, row, 0, unroll=False)


def _make_sc_agg():
  out_type = [jax.ShapeDtypeStruct((NPAD, DH), jnp.float32)]

  scratch = [
      pltpu.VMEM((NB + NBUF, B), jnp.int32),
      pltpu.VMEM((NB, B), jnp.int32),
      [pltpu.VMEM((B, QW), jnp.float32) for _ in range(NBUF)],
      pltpu.VMEM((ZB, QW), jnp.float32),
      pltpu.VMEM_SHARED((NPAD, QW), jnp.float32),
      pltpu.VMEM_SHARED((NPAD, QW), jnp.float32),
      [pltpu.SemaphoreType.DMA for _ in range(NBUF)],
      [pltpu.SemaphoreType.DMA for _ in range(NBUF)],
  ]

  def body(m4_hbm, src_hbm, dst_hbm, out_hbm,
           srcv, dstv, bufs, zbuf, mS, aggS, gsems, ssems):
    c = lax.axis_index("c")
    s = lax.axis_index("s")

    pltpu.sync_copy(src_hbm.at[s], srcv)
    pltpu.sync_copy(dst_hbm.at[s], dstv)
    _fill_rows(zbuf, ZB, QW, 0.0)

    def gather(j, bi):
      pltpu.make_async_copy(mS.at[srcv.at[j]], bufs[bi], gsems[bi]).start()

    def gwait(j, bi):
      pltpu.make_async_copy(mS.at[srcv.at[j]], bufs[bi], gsems[bi]).wait()

    def sstart(j, bi):
      pltpu.async_copy(bufs[bi], aggS.at[dstv.at[j]], ssems[bi], add=True)

    def swait(bi):
      pltpu.make_async_copy(bufs[bi], aggS.at[dstv.at[0]], ssems[bi]).wait()

    def stepj(j, bi, first):
      gwait(j, bi)
      sstart(j, bi)
      sw = (bi + KLA) % NBUF
      if not first:
        swait(sw)
      gather(j + KLA, sw)

    for p in range(2):
      qc = 2 * p + c
      for t in range(ROWS_PER_TILE // ZB):
        pltpu.sync_copy(zbuf, aggS.at[pl.ds(s * ROWS_PER_TILE + t * ZB, ZB)])
      pltpu.sync_copy(
          m4_hbm.at[pl.ds(s * ROWS_PER_TILE, ROWS_PER_TILE),
                    pl.ds(qc * QW, QW)],
          mS.at[pl.ds(s * ROWS_PER_TILE, ROWS_PER_TILE)])
      plsc.subcore_barrier()

      for j in range(KLA):
        gather(j, j)
      for j in range(NBUF):
        stepj(j, j, j < NBUF - KLA)

      def step(g, _):
        for bi in range(NBUF):
          stepj(g * NBUF + bi, bi, False)
        return 0

      lax.fori_loop(1, NB // NBUF, step, 0, unroll=False)
      for i in range(KLA):
        gwait(NB + i, i)
      swait(NBUF - 2)
      swait(NBUF - 1)
      plsc.subcore_barrier()

      for t in range(RCH):
        r = (s * RCH + t) * B
        pltpu.sync_copy(aggS.at[pl.ds(r, B)],
                        out_hbm.at[pl.ds(r, B), pl.ds(qc * QW, QW)])

  return pl.kernel(
      body, out_type=out_type, mesh=_sc_mesh, scratch_types=scratch,
      compiler_params=pltpu.CompilerParams(use_tc_tiling_on_sc=False))


NBD = EP // (NC * NS * B)


def _make_sc_deg():
  out_type = [jax.ShapeDtypeStruct((NC, NPAD, 16), jnp.float32)]
  scratch = [
      pltpu.VMEM((NBD, B), jnp.int32),
      pltpu.VMEM((B, 16), jnp.float32),
      pltpu.VMEM((B, 16), jnp.float32),
      pltpu.VMEM_SHARED((NPAD, 16), jnp.float32),
  ]

  def body(dst_hbm, out_hbm, dstv, onesv, z16, degS):
    c = lax.axis_index("c")
    s = lax.axis_index("s")
    w = c * NS + s

    pltpu.sync_copy(dst_hbm.at[w], dstv)
    _fill_rows(onesv, B, 16, 1.0)
    _fill_rows(z16, B, 16, 0.0)
    for t in range(RCH):
      pltpu.sync_copy(z16, degS.at[pl.ds((s * RCH + t) * B, B)])
    plsc.subcore_barrier()

    def step(j, _):
      pltpu.sync_copy(onesv, degS.at[dstv.at[j]], add=True)
      return 0

    lax.fori_loop(0, NBD, step, 0, unroll=False)
    plsc.subcore_barrier()

    for t in range(RCH):
      r = (s * RCH + t) * B
      pltpu.sync_copy(degS.at[pl.ds(r, B)], out_hbm.at[c, pl.ds(r, B)])

  return pl.kernel(
      body, out_type=out_type, mesh=_sc_mesh, scratch_types=scratch,
      compiler_params=pltpu.CompilerParams(use_tc_tiling_on_sc=False))


_sc_agg = _make_sc_agg()
_sc_deg = _make_sc_deg()

BLK = 1000
NBLK = N // BLK


def _tc1_body(x_ref, wn_ref, ws_ref, m_ref, s_ref):
  xb = x_ref[...]
  m_ref[...] = jnp.dot(xb, wn_ref[...], preferred_element_type=jnp.float32)
  s_ref[...] = jnp.dot(xb, ws_ref[...], preferred_element_type=jnp.float32)


def _tc1(x, wn, ws):
  return pl.pallas_call(
      _tc1_body,
      grid=(NBLK,),
      in_specs=[
          pl.BlockSpec((BLK, DH), lambda i: (i, 0)),
          pl.BlockSpec((DH, DH), lambda i: (0, 0)),
          pl.BlockSpec((DH, DH), lambda i: (0, 0)),
      ],
      out_specs=[
          pl.BlockSpec((BLK, DH), lambda i: (i, 0)),
          pl.BlockSpec((BLK, DH), lambda i: (i, 0)),
      ],
      out_shape=[
          jax.ShapeDtypeStruct((NPAD, DH), jnp.float32),
          jax.ShapeDtypeStruct((N, DH), jnp.float32),
      ],
  )(x, wn, ws)


def _combine(s_ref, a_ref, d_ref):
  deg = d_ref[0, :, 0:1] + d_ref[1, :, 0:1]
  h = s_ref[...] + a_ref[...] / jnp.maximum(deg, 1.0)
  return jnp.where(h > 0, h, jnp.exp(h) - 1.0)


def _tc2_body(s_ref, a_ref, d_ref, wn_ref, ws_ref, m_ref, o_ref):
  h = _combine(s_ref, a_ref, d_ref)
  m_ref[...] = jnp.dot(h, wn_ref[...], preferred_element_type=jnp.float32)
  o_ref[...] = jnp.dot(h, ws_ref[...], preferred_element_type=jnp.float32)


def _tc2(s1, agg1, degw, wn, ws):
  return pl.pallas_call(
      _tc2_body,
      grid=(NBLK,),
      in_specs=[
          pl.BlockSpec((BLK, DH), lambda i: (i, 0)),
          pl.BlockSpec((BLK, DH), lambda i: (i, 0)),
          pl.BlockSpec((NC, BLK, 16), lambda i: (0, i, 0)),
          pl.BlockSpec((DH, DH), lambda i: (0, 0)),
          pl.BlockSpec((DH, DH), lambda i: (0, 0)),
      ],
      out_specs=[
          pl.BlockSpec((BLK, DH), lambda i: (i, 0)),
          pl.BlockSpec((BLK, DH), lambda i: (i, 0)),
      ],
      out_shape=[
          jax.ShapeDtypeStruct((NPAD, DH), jnp.float32),
          jax.ShapeDtypeStruct((N, DH), jnp.float32),
      ],
  )(s1, agg1, degw, wn, ws)


def _tc3_body(gid_ref, s_ref, a_ref, d_ref, emb_ref, sent_ref):
  i = pl.program_id(0)
  emb = _combine(s_ref, a_ref, d_ref)
  emb_ref[...] = emb

  @pl.when(i == 0)
  def _():
    sent_ref[...] = jnp.full((G, DH), -jnp.inf, jnp.float32)

  gid = gid_ref[...]
  gmin = jnp.min(gid)
  gmax = jnp.max(gid)

  def gbody(g, _):
    v = jnp.max(jnp.where(gid == g, emb, -jnp.inf), axis=0, keepdims=True)
    cur = sent_ref[pl.ds(g, 1), :]
    sent_ref[pl.ds(g, 1), :] = jnp.maximum(cur, v)
    return 0

  lax.fori_loop(gmin, gmax + 1, gbody, 0, unroll=False)


def _tc3(gid2d, s2, agg2, degw):
  return pl.pallas_call(
      _tc3_body,
      grid=(NBLK,),
      in_specs=[
          pl.BlockSpec((BLK, 1), lambda i: (i, 0)),
          pl.BlockSpec((BLK, DH), lambda i: (i, 0)),
          pl.BlockSpec((BLK, DH), lambda i: (i, 0)),
          pl.BlockSpec((NC, BLK, 16), lambda i: (0, i, 0)),
      ],
      out_specs=[
          pl.BlockSpec((BLK, DH), lambda i: (i, 0)),
          pl.BlockSpec((G, DH), lambda i: (0, 0)),
      ],
      out_shape=[
          jax.ShapeDtypeStruct((N, DH), jnp.float32),
          jax.ShapeDtypeStruct((G, DH), jnp.float32),
      ],
  )(gid2d, s2, agg2, degw)


def _tc4_body(e1_ref, e2_ref, sent_ref, emb_ref, tgt_ref, wl_ref, bl_ref,
              pred_ref, loss_ref):
  emb = emb_ref[...]
  nodes = lax.broadcasted_iota(jnp.int32, (G, N), 1)
  oh1 = (nodes == e1_ref[...]).astype(jnp.float32)
  oh2 = (nodes == e2_ref[...]).astype(jnp.float32)
  e1rows = jnp.dot(oh1, emb, preferred_element_type=jnp.float32)
  e2rows = jnp.dot(oh2, emb, preferred_element_type=jnp.float32)
  h = jnp.concatenate([sent_ref[...], e1rows, e2rows], axis=1)
  logits = jnp.dot(h, wl_ref[...], preferred_element_type=jnp.float32)
  logits = logits + bl_ref[...]
  mx = jnp.max(logits, axis=1, keepdims=True)
  lse = mx + jnp.log(jnp.sum(jnp.exp(logits - mx), axis=1, keepdims=True))
  logp = logits - lse
  cols = lax.broadcasted_iota(jnp.int32, (G, C), 1)
  onehot = cols == tgt_ref[...]
  picked = jnp.sum(jnp.where(onehot, logp, 0.0), axis=1)
  ce = -jnp.mean(picked)
  loss = ce + 0.003 * jnp.mean(jnp.sum(h * h, axis=1))
  loss_ref[...] = jnp.reshape(loss, (1, 1))
  pred = jnp.min(jnp.where(logits == mx, cols, C), axis=1)
  pred_ref[...] = pred.reshape(G, 1)


def _tc4(e1_idx, e2_idx, sent, emb, tgt2d, wl, bl2d):
  return pl.pallas_call(
      _tc4_body,
      in_specs=[
          pl.BlockSpec((G, 1), lambda: (0, 0)),
          pl.BlockSpec((G, 1), lambda: (0, 0)),
          pl.BlockSpec((G, DH), lambda: (0, 0)),
          pl.BlockSpec((N, DH), lambda: (0, 0)),
          pl.BlockSpec((G, 1), lambda: (0, 0)),
          pl.BlockSpec((3 * DH, C), lambda: (0, 0)),
          pl.BlockSpec((1, C), lambda: (0, 0)),
      ],
      out_specs=[
          pl.BlockSpec((G, 1), lambda: (0, 0)),
          pl.BlockSpec((1, 1), lambda: (0, 0)),
      ],
      out_shape=[
          jax.ShapeDtypeStruct((G, 1), jnp.int32),
          jax.ShapeDtypeStruct((1, 1), jnp.float32),
      ],
  )(e1_idx, e2_idx, sent, emb, tgt2d, wl, bl2d)


def kernel(x, edge_index, graph_ids, e1_idx, e2_idx, tgt,
           W_self1, W_neigh1, W_self2, W_neigh2, W_lin, b_lin):
  src = edge_index[0]
  dst = edge_index[1]
  pad = EP - E
  srcp = jnp.concatenate([src, jnp.zeros((pad,), jnp.int32)])
  dstp = jnp.concatenate([dst, jnp.full((pad,), N, jnp.int32)])
  src3 = jnp.concatenate(
      [srcp.reshape(NS, NB, B),
       jnp.zeros((NS, NBUF, B), jnp.int32)], axis=1)
  dst3 = dstp.reshape(NS, NB, B)
  dst4 = dstp.reshape(NC * NS, NBD, B)
  gid2d = graph_ids.reshape(N, 1)
  tgt2d = tgt.reshape(G, 1).astype(jnp.int32)
  bl2d = b_lin.reshape(1, C)

  m1, s1 = _tc1(x, W_neigh1, W_self1)
  degw, = _sc_deg(dst4)
  agg1, = _sc_agg(m1, src3, dst3)
  m2, s2 = _tc2(s1, agg1, degw, W_neigh2, W_self2)
  agg2, = _sc_agg(m2, src3, dst3)
  emb, sent = _tc3(gid2d, s2, agg2, degw)
  pred2, loss2 = _tc4(e1_idx.reshape(G, 1), e2_idx.reshape(G, 1),
                      sent, emb, tgt2d, W_lin, bl2d)
  return pred2[:, 0], loss2[0, 0]
